# baseline (speedup 1.0000x reference)
import functools

import jax
import jax.numpy as jnp
from jax import lax
from jax.experimental import pallas as pl
from jax.experimental.pallas import tpu as pltpu
from jax.experimental.pallas import tpu_sc as plsc

N = 10000
E = 320000
D_IN, HID, OUT = 128, 64, 128

NC, NS, LANES = 2, 16, 16
NW = NC * NS
BLK = 128
NBLK = 80
EPW = NBLK * BLK
E_PAD = NW * EPW
N_PAD = 10240
RPS = N_PAD // NS
CNT_W = 16
ZR = 128

TC_BLK = 1000

_PERM = [32 * (q // 32) + (q % 32) // 2 + 16 * (q % 2) for q in range(HID)]


NBUF = 4


def _seg_body(with_counts, *refs):
    if with_counts:
        (vals, srcs, dsts, out, cnt_out,
         src_v, dst_v, bf0, bf1, bf2, bf3, f0, f1, f2, f3, hist_v, zero_v,
         acc_sh, sem_i, g0, g1, g2, g3, s0, s1, s2, s3) = refs
    else:
        (vals, srcs, dsts, out,
         src_v, dst_v, bf0, bf1, bf2, bf3, f0, f1, f2, f3, zero_v,
         acc_sh, sem_i, g0, g1, g2, g3, s0, s1, s2, s3) = refs
    bf = [bf0, bf1, bf2, bf3]
    f32 = [f0, f1, f2, f3]
    gsem = [g0, g1, g2, g3]
    ssem = [s0, s1, s2, s3]
    c = lax.axis_index("c")
    s = lax.axis_index("s")
    wid = s * NC + c

    pltpu.async_copy(srcs.at[wid], src_v, sem_i)
    pltpu.async_copy(dsts.at[wid], dst_v, sem_i)

    zvec = jnp.zeros((LANES,), jnp.float32)

    @pl.loop(0, ZR)
    def _(i):
        for j in range(0, HID, LANES):
            zero_v[i, pl.ds(j, LANES)] = zvec

    if with_counts:
        @pl.loop(0, N_PAD, step=LANES)
        def _(i):
            hist_v[pl.ds(i, LANES)] = zvec

    for k in range(RPS // ZR):
        r0 = s * RPS + k * ZR
        pltpu.sync_copy(zero_v, acc_sh.at[pl.ds(r0, ZR)])
    plsc.subcore_barrier()

    pltpu.make_async_copy(srcs.at[0], src_v, sem_i).wait()
    pltpu.make_async_copy(dsts.at[0], dst_v, sem_i).wait()

    def fire(g, buf, sem):
        pltpu.async_copy(vals.at[src_v.at[g]], buf, sem)

    def drain(buf, sem):
        pltpu.make_async_copy(vals.at[pl.ds(0, BLK)], buf, sem).wait()

    def convert(bf_buf, f32_buf):
        @plsc.parallel_loop(0, BLK, unroll=8)
        def _(r):
            for j in range(0, HID, 2 * LANES):
                pair = bf_buf[r, pl.ds(j, 2 * LANES)]
                lo, hi = plsc.unpack(pair, format=plsc.PackFormat.INTERLEAVED,
                                     preferred_element_type=jnp.float32)
                f32_buf[r, pl.ds(j, LANES)] = lo
                f32_buf[r, pl.ds(j + LANES, LANES)] = hi

    def sc_fire(g, f32_buf, sem):
        pltpu.async_copy(f32_buf, acc_sh.at[dst_v.at[g]], sem, add=True)

    def sc_drain(f32_buf, sem):
        pltpu.make_async_copy(out.at[0].at[pl.ds(0, BLK)], f32_buf, sem).wait()

    ones16 = jnp.ones((LANES,), jnp.float32)

    def count(g):
        if with_counts:
            for j in range(0, BLK, LANES):
                idx = dst_v[g, pl.ds(j, LANES)]
                plsc.addupdate_scatter(hist_v, [idx], ones16)

    for k in range(NBUF - 1):
        fire(k, bf[k], gsem[k])

    @pl.loop(0, NBLK, step=NBUF)
    def _(g):
        for k in range(NBUF):
            blk_id = g + k
            drain(bf[k], gsem[k])

            @pl.when(blk_id >= NBUF)
            def _():
                sc_drain(f32[k], ssem[k])
            convert(bf[k], f32[k])
            sc_fire(blk_id, f32[k], ssem[k])
            count(blk_id)

            @pl.when(blk_id + NBUF - 1 < NBLK)
            def _():
                fire(blk_id + NBUF - 1, bf[(k + NBUF - 1) % NBUF],
                     gsem[(k + NBUF - 1) % NBUF])

    for k in range(NBUF):
        sc_drain(f32[k], ssem[k])

    if with_counts:
        pltpu.sync_copy(hist_v, cnt_out.at[c].at[s])

    plsc.subcore_barrier()

    r0 = s * RPS
    pltpu.sync_copy(acc_sh.at[pl.ds(r0, RPS)], out.at[c].at[pl.ds(r0, RPS)])


def _make_seg(with_counts):
    mesh = plsc.VectorSubcoreMesh(core_axis_name="c", subcore_axis_name="s")
    out_type = [jax.ShapeDtypeStruct((NC, N_PAD, HID), jnp.float32)]
    scratch = [
        pltpu.VMEM((NBLK, BLK), jnp.int32),
        pltpu.VMEM((NBLK, BLK), jnp.int32),
    ]
    scratch += [pltpu.VMEM((BLK, HID), jnp.bfloat16) for _ in range(NBUF)]
    scratch += [pltpu.VMEM((BLK, HID), jnp.float32) for _ in range(NBUF)]
    if with_counts:
        out_type.append(jax.ShapeDtypeStruct((NC, NS, N_PAD), jnp.float32))
        scratch.append(pltpu.VMEM((N_PAD,), jnp.float32))
    scratch += [
        pltpu.VMEM((ZR, HID), jnp.float32),
        pltpu.VMEM_SHARED((N_PAD, HID), jnp.float32),
    ]
    scratch += [pltpu.SemaphoreType.DMA] * (1 + 2 * NBUF)
    return pl.kernel(
        functools.partial(_seg_body, with_counts),
        out_type=out_type if with_counts else out_type[0],
        mesh=mesh,
        scratch_types=scratch,
        compiler_params=pltpu.CompilerParams(use_tc_tiling_on_sc=False,
                                             needs_layout_passes=False),
    )


_seg_with_counts = _make_seg(True)
_seg_plain = _make_seg(False)


def _interleave_cols(a):
    r = a.shape[0]
    g = a.reshape(r, HID // 32, 2, 16)
    g = jnp.swapaxes(g, 2, 3)
    return g.reshape(r, HID).astype(jnp.bfloat16)


def _tc1_body(x_ref, w_ref, b_ref, y_ref, xr_ref):
    prod = jnp.dot(x_ref[...], w_ref[...], preferred_element_type=jnp.float32)
    y_ref[...] = _interleave_cols(prod[:, :HID])
    xr_ref[...] = prod[:, HID:] + b_ref[...]


def _tc2_body(s0_ref, s1_ref, c_ref, xr_ref, h_ref, hb_ref, inv_ref):
    cnt = jnp.sum(c_ref[...], axis=1)
    inv = (1.0 / jnp.maximum(cnt, 1.0))[:, None]
    aggr = (s0_ref[...] + s1_ref[...]) * inv
    h = jnp.maximum(aggr + xr_ref[...], 0.0)
    h_ref[...] = h
    hb_ref[...] = _interleave_cols(h)
    inv_ref[...] = jnp.broadcast_to(inv, (TC_BLK, CNT_W))


def _tc3_body(s0_ref, s1_ref, inv_ref, h_ref, w_ref, b_ref, out_ref):
    aggr = (s0_ref[...] + s1_ref[...]) * inv_ref[:, 0:1]
    z = jnp.concatenate([aggr, h_ref[...]], axis=1)
    out_ref[...] = jnp.dot(z, w_ref[...], preferred_element_type=jnp.float32) + b_ref[...]


def _row_spec(width):
    return pl.BlockSpec((TC_BLK, width), lambda i: (i, 0))


def _full_spec(shape):
    return pl.BlockSpec(shape, lambda i: tuple(0 for _ in shape))


_GRID = N // TC_BLK

_tc1 = pl.pallas_call(
    _tc1_body,
    grid=(_GRID,),
    in_specs=[_row_spec(D_IN), _full_spec((D_IN, 2 * HID)), _full_spec((1, HID))],
    out_specs=[_row_spec(HID), _row_spec(HID)],
    out_shape=[jax.ShapeDtypeStruct((N, HID), jnp.bfloat16),
               jax.ShapeDtypeStruct((N, HID), jnp.float32)],
)

_tc2 = pl.pallas_call(
    _tc2_body,
    grid=(_GRID,),
    in_specs=[_row_spec(HID), _row_spec(HID), _row_spec(NW), _row_spec(HID)],
    out_specs=[_row_spec(HID), _row_spec(HID), _row_spec(CNT_W)],
    out_shape=[jax.ShapeDtypeStruct((N, HID), jnp.float32),
               jax.ShapeDtypeStruct((N, HID), jnp.bfloat16),
               jax.ShapeDtypeStruct((N, CNT_W), jnp.float32)],
)

_tc3 = pl.pallas_call(
    _tc3_body,
    grid=(_GRID,),
    in_specs=[_row_spec(HID), _row_spec(HID), _row_spec(CNT_W), _row_spec(HID),
              _full_spec((2 * HID, OUT)), _full_spec((1, OUT))],
    out_specs=_row_spec(OUT),
    out_shape=jax.ShapeDtypeStruct((N, OUT), jnp.float32),
)


def kernel(x, edge_index, W1_l, b1, W1_r, W2_l, b2, W2_r):
    w1 = jnp.concatenate([W1_l.T, W1_r.T], axis=1)
    w2 = jnp.concatenate([W2_l.T, W2_r.T], axis=0)
    pad = E_PAD - E
    srcs = jnp.concatenate([edge_index[0], jnp.zeros((pad,), jnp.int32)])
    dsts = jnp.concatenate([edge_index[1], jnp.full((pad,), N, jnp.int32)])
    srcs = srcs.reshape(NW, NBLK, BLK)
    dsts = dsts.reshape(NW, NBLK, BLK)
    y1b, xr1 = _tc1(x, w1, b1[None, :])
    s1p, cntp = _seg_with_counts(y1b, srcs, dsts)
    cnt_t = cntp.reshape(NW, N_PAD)[:, :N].T
    h, hb, inv = _tc2(s1p[0, :N], s1p[1, :N], cnt_t, xr1)
    s2p = _seg_plain(hb, srcs, dsts)
    out = _tc3(s2p[0, :N], s2p[1, :N], inv, h, w2, b2[None, :])
    return out


# speedup vs baseline: 1.0102x; 1.0102x over previous
import functools

import jax
import jax.numpy as jnp
from jax import lax
from jax.experimental import pallas as pl
from jax.experimental.pallas import tpu as pltpu
from jax.experimental.pallas import tpu_sc as plsc

N = 10000
E = 320000
D_IN, HID, OUT = 128, 64, 128

NC, NS, LANES = 2, 16, 16
NW = NC * NS
BLK = 128
NBLK = 80
EPW = NBLK * BLK
E_PAD = NW * EPW
N_PAD = 10240
RPS = N_PAD // NS
CNT_W = 16
ZR = 128

TC_BLK = 1000

_PERM = [32 * (q // 32) + (q % 32) // 2 + 16 * (q % 2) for q in range(HID)]


NBUF = 4


def _seg_body(with_counts, *refs):
    if with_counts:
        (vals, srcs, dsts, out, cnt_out,
         src_v, dst_v, bf0, bf1, bf2, bf3, f0, f1, f2, f3, hist_v, zero_v,
         acc_sh, sem_i, g0, g1, g2, g3, s0, s1, s2, s3) = refs
    else:
        (vals, srcs, dsts, out,
         src_v, dst_v, bf0, bf1, bf2, bf3, f0, f1, f2, f3, zero_v,
         acc_sh, sem_i, g0, g1, g2, g3, s0, s1, s2, s3) = refs
    bf = [bf0, bf1, bf2, bf3]
    f32 = [f0, f1, f2, f3]
    gsem = [g0, g1, g2, g3]
    ssem = [s0, s1, s2, s3]
    c = lax.axis_index("c")
    s = lax.axis_index("s")
    wid = s * NC + c

    pltpu.async_copy(srcs.at[wid], src_v, sem_i)
    pltpu.async_copy(dsts.at[wid], dst_v, sem_i)

    zvec = jnp.zeros((LANES,), jnp.float32)

    @pl.loop(0, ZR)
    def _(i):
        for j in range(0, HID, LANES):
            zero_v[i, pl.ds(j, LANES)] = zvec

    if with_counts:
        @pl.loop(0, N_PAD, step=LANES)
        def _(i):
            hist_v[pl.ds(i, LANES)] = zvec

    for k in range(RPS // ZR):
        r0 = s * RPS + k * ZR
        pltpu.sync_copy(zero_v, acc_sh.at[pl.ds(r0, ZR)])
    plsc.subcore_barrier()

    pltpu.make_async_copy(srcs.at[0], src_v, sem_i).wait()
    pltpu.make_async_copy(dsts.at[0], dst_v, sem_i).wait()

    def fire(g, buf, sem):
        pltpu.async_copy(vals.at[src_v.at[g]], buf, sem)

    def drain(buf, sem):
        pltpu.make_async_copy(vals.at[pl.ds(0, BLK)], buf, sem).wait()

    def convert(bf_buf, f32_buf):
        @plsc.parallel_loop(0, BLK, unroll=4)
        def _(r):
            for j in range(0, HID, 2 * LANES):
                pair = bf_buf[r, pl.ds(j, 2 * LANES)]
                lo, hi = plsc.unpack(pair, format=plsc.PackFormat.INTERLEAVED,
                                     preferred_element_type=jnp.float32)
                f32_buf[r, pl.ds(j, LANES)] = lo
                f32_buf[r, pl.ds(j + LANES, LANES)] = hi

    def sc_fire(g, f32_buf, sem):
        pltpu.async_copy(f32_buf, acc_sh.at[dst_v.at[g]], sem, add=True)

    def sc_drain(f32_buf, sem):
        pltpu.make_async_copy(out.at[0].at[pl.ds(0, BLK)], f32_buf, sem).wait()

    ones16 = jnp.ones((LANES,), jnp.float32)

    def count(g):
        if with_counts:
            for j in range(0, BLK, LANES):
                idx = dst_v[g, pl.ds(j, LANES)]
                plsc.addupdate_scatter(hist_v, [idx], ones16)

    for k in range(NBUF - 1):
        fire(k, bf[k], gsem[k])

    @pl.loop(0, NBLK, step=NBUF)
    def _(g):
        for k in range(NBUF):
            blk_id = g + k
            drain(bf[k], gsem[k])

            @pl.when(blk_id >= NBUF)
            def _():
                sc_drain(f32[k], ssem[k])
            convert(bf[k], f32[k])
            sc_fire(blk_id, f32[k], ssem[k])
            count(blk_id)

            @pl.when(blk_id + NBUF - 1 < NBLK)
            def _():
                fire(blk_id + NBUF - 1, bf[(k + NBUF - 1) % NBUF],
                     gsem[(k + NBUF - 1) % NBUF])

    for k in range(NBUF):
        sc_drain(f32[k], ssem[k])

    if with_counts:
        pltpu.sync_copy(hist_v, cnt_out.at[c].at[s])

    plsc.subcore_barrier()

    r0 = s * RPS
    pltpu.sync_copy(acc_sh.at[pl.ds(r0, RPS)], out.at[c].at[pl.ds(r0, RPS)])


def _make_seg(with_counts):
    mesh = plsc.VectorSubcoreMesh(core_axis_name="c", subcore_axis_name="s")
    out_type = [jax.ShapeDtypeStruct((NC, N_PAD, HID), jnp.float32)]
    scratch = [
        pltpu.VMEM((NBLK, BLK), jnp.int32),
        pltpu.VMEM((NBLK, BLK), jnp.int32),
    ]
    scratch += [pltpu.VMEM((BLK, HID), jnp.bfloat16) for _ in range(NBUF)]
    scratch += [pltpu.VMEM((BLK, HID), jnp.float32) for _ in range(NBUF)]
    if with_counts:
        out_type.append(jax.ShapeDtypeStruct((NC, NS, N_PAD), jnp.float32))
        scratch.append(pltpu.VMEM((N_PAD,), jnp.float32))
    scratch += [
        pltpu.VMEM((ZR, HID), jnp.float32),
        pltpu.VMEM_SHARED((N_PAD, HID), jnp.float32),
    ]
    scratch += [pltpu.SemaphoreType.DMA] * (1 + 2 * NBUF)
    return pl.kernel(
        functools.partial(_seg_body, with_counts),
        out_type=out_type if with_counts else out_type[0],
        mesh=mesh,
        scratch_types=scratch,
        compiler_params=pltpu.CompilerParams(use_tc_tiling_on_sc=False,
                                             needs_layout_passes=False),
    )


_seg_with_counts = _make_seg(True)
_seg_plain = _make_seg(False)


def _interleave_cols(a):
    r = a.shape[0]
    g = a.reshape(r, HID // 32, 2, 16)
    g = jnp.swapaxes(g, 2, 3)
    return g.reshape(r, HID).astype(jnp.bfloat16)


def _tc1_body(x_ref, w_ref, b_ref, y_ref, xr_ref):
    prod = jnp.dot(x_ref[...], w_ref[...], preferred_element_type=jnp.float32)
    y_ref[...] = _interleave_cols(prod[:, :HID])
    xr_ref[...] = prod[:, HID:] + b_ref[...]


def _tc2_body(s0_ref, s1_ref, c_ref, xr_ref, h_ref, hb_ref, inv_ref):
    cnt = jnp.sum(c_ref[...], axis=1)
    inv = (1.0 / jnp.maximum(cnt, 1.0))[:, None]
    aggr = (s0_ref[...] + s1_ref[...]) * inv
    h = jnp.maximum(aggr + xr_ref[...], 0.0)
    h_ref[...] = h
    hb_ref[...] = _interleave_cols(h)
    inv_ref[...] = jnp.broadcast_to(inv, (TC_BLK, CNT_W))


def _tc3_body(s0_ref, s1_ref, inv_ref, h_ref, w_ref, b_ref, out_ref):
    aggr = (s0_ref[...] + s1_ref[...]) * inv_ref[:, 0:1]
    z = jnp.concatenate([aggr, h_ref[...]], axis=1)
    out_ref[...] = jnp.dot(z, w_ref[...], preferred_element_type=jnp.float32) + b_ref[...]


def _row_spec(width):
    return pl.BlockSpec((TC_BLK, width), lambda i: (i, 0))


def _full_spec(shape):
    return pl.BlockSpec(shape, lambda i: tuple(0 for _ in shape))


_GRID = N // TC_BLK

_tc1 = pl.pallas_call(
    _tc1_body,
    grid=(_GRID,),
    in_specs=[_row_spec(D_IN), _full_spec((D_IN, 2 * HID)), _full_spec((1, HID))],
    out_specs=[_row_spec(HID), _row_spec(HID)],
    out_shape=[jax.ShapeDtypeStruct((N, HID), jnp.bfloat16),
               jax.ShapeDtypeStruct((N, HID), jnp.float32)],
)

_tc2 = pl.pallas_call(
    _tc2_body,
    grid=(_GRID,),
    in_specs=[_row_spec(HID), _row_spec(HID), _row_spec(NW), _row_spec(HID)],
    out_specs=[_row_spec(HID), _row_spec(HID), _row_spec(CNT_W)],
    out_shape=[jax.ShapeDtypeStruct((N, HID), jnp.float32),
               jax.ShapeDtypeStruct((N, HID), jnp.bfloat16),
               jax.ShapeDtypeStruct((N, CNT_W), jnp.float32)],
)

_tc3 = pl.pallas_call(
    _tc3_body,
    grid=(_GRID,),
    in_specs=[_row_spec(HID), _row_spec(HID), _row_spec(CNT_W), _row_spec(HID),
              _full_spec((2 * HID, OUT)), _full_spec((1, OUT))],
    out_specs=_row_spec(OUT),
    out_shape=jax.ShapeDtypeStruct((N, OUT), jnp.float32),
)


def kernel(x, edge_index, W1_l, b1, W1_r, W2_l, b2, W2_r):
    w1 = jnp.concatenate([W1_l.T, W1_r.T], axis=1)
    w2 = jnp.concatenate([W2_l.T, W2_r.T], axis=0)
    pad = E_PAD - E
    srcs = jnp.concatenate([edge_index[0], jnp.zeros((pad,), jnp.int32)])
    dsts = jnp.concatenate([edge_index[1], jnp.full((pad,), N, jnp.int32)])
    srcs = srcs.reshape(NW, NBLK, BLK)
    dsts = dsts.reshape(NW, NBLK, BLK)
    y1b, xr1 = _tc1(x, w1, b1[None, :])
    s1p, cntp = _seg_with_counts(y1b, srcs, dsts)
    cnt_t = cntp.reshape(NW, N_PAD)[:, :N].T
    h, hb, inv = _tc2(s1p[0, :N], s1p[1, :N], cnt_t, xr1)
    s2p = _seg_plain(hb, srcs, dsts)
    out = _tc3(s2p[0, :N], s2p[1, :N], inv, h, w2, b2[None, :])
    return out


# speedup vs baseline: 1.0383x; 1.0278x over previous
import functools

import jax
import jax.numpy as jnp
from jax import lax
from jax.experimental import pallas as pl
from jax.experimental.pallas import tpu as pltpu
from jax.experimental.pallas import tpu_sc as plsc

N = 10000
E = 320000
D_IN, HID, OUT = 128, 64, 128

NC, NS, LANES = 2, 16, 16
NW = NC * NS
BLK = 128
NBLK = 80
EPW = NBLK * BLK
E_PAD = NW * EPW
N_PAD = 10240
RPS = N_PAD // NS
CNT_W = 16
ZR = 128

TC_BLK = 1000

_PERM = [32 * (q // 32) + (q % 32) // 2 + 16 * (q % 2) for q in range(HID)]


NBUF = 4


def _seg_body(with_counts, *refs):
    if with_counts:
        (vals, srcs, dsts, out, cnt_out,
         src_v, dst_v, bf0, bf1, bf2, bf3, f0, f1, f2, f3, hist_v, zero_v,
         acc_sh, sem_i, g0, g1, g2, g3, s0, s1, s2, s3) = refs
    else:
        (vals, srcs, dsts, out,
         src_v, dst_v, bf0, bf1, bf2, bf3, f0, f1, f2, f3, zero_v,
         acc_sh, sem_i, g0, g1, g2, g3, s0, s1, s2, s3) = refs
    bf = [bf0, bf1, bf2, bf3]
    f32 = [f0, f1, f2, f3]
    gsem = [g0, g1, g2, g3]
    ssem = [s0, s1, s2, s3]
    c = lax.axis_index("c")
    s = lax.axis_index("s")
    wid = s * NC + c

    pltpu.async_copy(srcs.at[wid], src_v, sem_i)
    pltpu.async_copy(dsts.at[wid], dst_v, sem_i)

    zvec = jnp.zeros((LANES,), jnp.float32)

    @pl.loop(0, ZR)
    def _(i):
        for j in range(0, HID, LANES):
            zero_v[i, pl.ds(j, LANES)] = zvec

    if with_counts:
        @pl.loop(0, N_PAD, step=LANES)
        def _(i):
            hist_v[pl.ds(i, LANES)] = zvec

    for k in range(RPS // ZR):
        r0 = s * RPS + k * ZR
        pltpu.sync_copy(zero_v, acc_sh.at[pl.ds(r0, ZR)])
    plsc.subcore_barrier()

    pltpu.make_async_copy(srcs.at[0], src_v, sem_i).wait()
    pltpu.make_async_copy(dsts.at[0], dst_v, sem_i).wait()

    def fire(g, buf, sem):
        pltpu.async_copy(vals.at[src_v.at[g]], buf, sem)

    def drain(buf, sem):
        pltpu.make_async_copy(vals.at[pl.ds(0, BLK)], buf, sem).wait()

    def convert(bf_buf, f32_buf):
        @plsc.parallel_loop(0, BLK, unroll=4)
        def _(r):
            for j in range(0, HID, 2 * LANES):
                pair = bf_buf[r, pl.ds(j, 2 * LANES)]
                lo, hi = plsc.unpack(pair, format=plsc.PackFormat.INTERLEAVED,
                                     preferred_element_type=jnp.float32)
                f32_buf[r, pl.ds(j, LANES)] = lo
                f32_buf[r, pl.ds(j + LANES, LANES)] = hi

    def sc_fire(g, f32_buf, sem):
        pltpu.async_copy(f32_buf, acc_sh.at[dst_v.at[g]], sem, add=True)

    def sc_drain(f32_buf, sem):
        pltpu.make_async_copy(out.at[0].at[pl.ds(0, BLK)], f32_buf, sem).wait()

    ones16 = jnp.ones((LANES,), jnp.float32)

    def count(g):
        if with_counts:
            for j in range(0, BLK, LANES):
                idx = dst_v[g, pl.ds(j, LANES)]
                plsc.addupdate_scatter(hist_v, [idx], ones16)

    for k in range(NBUF - 1):
        fire(k, bf[k], gsem[k])

    @pl.loop(0, NBLK, step=NBUF)
    def _(g):
        for k in range(NBUF):
            blk_id = g + k
            drain(bf[k], gsem[k])

            @pl.when(blk_id >= NBUF)
            def _():
                sc_drain(f32[k], ssem[k])
            convert(bf[k], f32[k])
            sc_fire(blk_id, f32[k], ssem[k])
            count(blk_id)

            @pl.when(blk_id + NBUF - 1 < NBLK)
            def _():
                fire(blk_id + NBUF - 1, bf[(k + NBUF - 1) % NBUF],
                     gsem[(k + NBUF - 1) % NBUF])

    for k in range(NBUF):
        sc_drain(f32[k], ssem[k])

    if with_counts:
        pltpu.sync_copy(hist_v, cnt_out.at[c].at[s])

    plsc.subcore_barrier()

    r0 = s * RPS
    pltpu.sync_copy(acc_sh.at[pl.ds(r0, RPS)], out.at[c].at[pl.ds(r0, RPS)])


def _make_seg(with_counts):
    mesh = plsc.VectorSubcoreMesh(core_axis_name="c", subcore_axis_name="s")
    out_type = [jax.ShapeDtypeStruct((NC, N_PAD, HID), jnp.float32)]
    scratch = [
        pltpu.VMEM((NBLK, BLK), jnp.int32),
        pltpu.VMEM((NBLK, BLK), jnp.int32),
    ]
    scratch += [pltpu.VMEM((BLK, HID), jnp.bfloat16) for _ in range(NBUF)]
    scratch += [pltpu.VMEM((BLK, HID), jnp.float32) for _ in range(NBUF)]
    if with_counts:
        out_type.append(jax.ShapeDtypeStruct((NC, NS, N_PAD), jnp.float32))
        scratch.append(pltpu.VMEM((N_PAD,), jnp.float32))
    scratch += [
        pltpu.VMEM((ZR, HID), jnp.float32),
        pltpu.VMEM_SHARED((N_PAD, HID), jnp.float32),
    ]
    scratch += [pltpu.SemaphoreType.DMA] * (1 + 2 * NBUF)
    return pl.kernel(
        functools.partial(_seg_body, with_counts),
        out_type=out_type if with_counts else out_type[0],
        mesh=mesh,
        scratch_types=scratch,
        compiler_params=pltpu.CompilerParams(use_tc_tiling_on_sc=False,
                                             needs_layout_passes=False),
    )


_seg_with_counts = _make_seg(True)
_seg_plain = _make_seg(False)


def _interleave_cols(a):
    r = a.shape[0]
    g = a.reshape(r, HID // 32, 2, 16)
    g = jnp.swapaxes(g, 2, 3)
    return g.reshape(r, HID).astype(jnp.bfloat16)


def _tc1_body(x_ref, w_ref, b_ref, y_ref, xr_ref):
    prod = jnp.dot(x_ref[...], w_ref[...], preferred_element_type=jnp.float32)
    y_ref[...] = _interleave_cols(prod[:, :HID])
    xr_ref[...] = prod[:, HID:] + b_ref[...]


def _tc2_body(s0_ref, s1_ref, c_ref, xr_ref, h_ref, hb_ref, inv_ref):
    cnt = jnp.sum(c_ref[...], axis=1)
    inv = (1.0 / jnp.maximum(cnt, 1.0))[:, None]
    aggr = (s0_ref[0] + s1_ref[0]) * inv
    h = jnp.maximum(aggr + xr_ref[...], 0.0)
    h_ref[...] = h
    hb_ref[...] = _interleave_cols(h)
    inv_ref[...] = jnp.broadcast_to(inv, (TC_BLK, CNT_W))


def _tc3_body(s0_ref, s1_ref, inv_ref, h_ref, w_ref, b_ref, out_ref):
    aggr = (s0_ref[0] + s1_ref[0]) * inv_ref[:, 0:1]
    z = jnp.concatenate([aggr, h_ref[...]], axis=1)
    out_ref[...] = jnp.dot(z, w_ref[...], preferred_element_type=jnp.float32) + b_ref[...]


def _row_spec(width):
    return pl.BlockSpec((TC_BLK, width), lambda i: (i, 0))


def _full_spec(shape):
    return pl.BlockSpec(shape, lambda i: tuple(0 for _ in shape))


_GRID = N // TC_BLK

_tc1 = pl.pallas_call(
    _tc1_body,
    grid=(_GRID,),
    in_specs=[_row_spec(D_IN), _full_spec((D_IN, 2 * HID)), _full_spec((1, HID))],
    out_specs=[_row_spec(HID), _row_spec(HID)],
    out_shape=[jax.ShapeDtypeStruct((N, HID), jnp.bfloat16),
               jax.ShapeDtypeStruct((N, HID), jnp.float32)],
)

def _part_spec(core):
    return pl.BlockSpec((1, TC_BLK, HID), lambda i, c=core: (c, i, 0))


_tc2 = pl.pallas_call(
    _tc2_body,
    grid=(_GRID,),
    in_specs=[_part_spec(0), _part_spec(1), _row_spec(NW), _row_spec(HID)],
    out_specs=[_row_spec(HID), _row_spec(HID), _row_spec(CNT_W)],
    out_shape=[jax.ShapeDtypeStruct((N, HID), jnp.float32),
               jax.ShapeDtypeStruct((N, HID), jnp.bfloat16),
               jax.ShapeDtypeStruct((N, CNT_W), jnp.float32)],
)

_tc3 = pl.pallas_call(
    _tc3_body,
    grid=(_GRID,),
    in_specs=[_part_spec(0), _part_spec(1), _row_spec(CNT_W), _row_spec(HID),
              _full_spec((2 * HID, OUT)), _full_spec((1, OUT))],
    out_specs=_row_spec(OUT),
    out_shape=jax.ShapeDtypeStruct((N, OUT), jnp.float32),
)


def kernel(x, edge_index, W1_l, b1, W1_r, W2_l, b2, W2_r):
    w1 = jnp.concatenate([W1_l.T, W1_r.T], axis=1)
    w2 = jnp.concatenate([W2_l.T, W2_r.T], axis=0)
    pad = E_PAD - E
    srcs = jnp.concatenate([edge_index[0], jnp.zeros((pad,), jnp.int32)])
    dsts = jnp.concatenate([edge_index[1], jnp.full((pad,), N, jnp.int32)])
    srcs = srcs.reshape(NW, NBLK, BLK)
    dsts = dsts.reshape(NW, NBLK, BLK)
    y1b, xr1 = _tc1(x, w1, b1[None, :])
    s1p, cntp = _seg_with_counts(y1b, srcs, dsts)
    cnt_t = cntp.reshape(NW, N_PAD)[:, :N].T
    h, hb, inv = _tc2(s1p, s1p, cnt_t, xr1)
    s2p = _seg_plain(hb, srcs, dsts)
    out = _tc3(s2p, s2p, inv, h, w2, b2[None, :])
    return out
